# baseline (speedup 1.0000x reference)
import functools

import jax
import jax.numpy as jnp
from jax import lax
from jax.experimental import pallas as pl
from jax.experimental.pallas import tpu as pltpu
from jax.experimental.pallas import tpu_sc as plsc

F32 = jnp.float32
I32 = jnp.int32

N_NODES = 10000
N_EDGES = 320000
EMB = 128
NUM_CLASS = 128
NUM_LAYER = 5
NUM_GRAPHS = 512

NC, NS = 2, 16
NW = NC * NS
NPAD = 10240
GPAD = 640
JUNK_ROW = NPAD - 1

CH = 128
NCHUNK = 80
NBUF = 2
EPT = NCHUNK * CH
EPAD = NW * EPT

RPT = NPAD // NW
SCH = 64
NSCH = RPT // SCH
NROWS = NPAD // NS
GROWS = GPAD // NS

_mesh = plsc.VectorSubcoreMesh(core_axis_name="c", subcore_axis_name="s")


def _zero_buf(buf, nrows):
    def body(i, _):
        buf[i // 8, pl.ds((i % 8) * 16, 16)] = jnp.zeros((16,), F32)
        return 0
    lax.fori_loop(0, nrows * 8, body, 0)


HCH = NCHUNK // 2


def _edge_body(r_hbm, src_hbm, dst_hbm, out_hbm, src_v, dst_v,
               b0, b1, agg_sh, s0, s1):
    bufs = (b0, b1)
    sems = (s0, s1)
    cid = lax.axis_index("c")
    sid = lax.axis_index("s")
    wid = cid * NS + sid

    _zero_buf(b0, CH)
    def zcopy(i, _):
        pltpu.sync_copy(b0, agg_sh.at[pl.ds(sid * NROWS + i * CH, CH)])
        return 0
    lax.fori_loop(0, NROWS // CH, zcopy, 0)
    plsc.subcore_barrier()

    for half in range(2):
        pltpu.sync_copy(src_hbm.at[wid, pl.ds(half * HCH, HCH)], src_v)
        pltpu.sync_copy(dst_hbm.at[wid, pl.ds(half * HCH, HCH)], dst_v)

        def body(jj, _):
            for b in range(2):
                j = jj * 2 + b
                @pl.when(j >= 2)
                def _():
                    pltpu.make_async_copy(bufs[b], agg_sh.at[dst_v.at[0]],
                                          sems[b]).wait()
                pltpu.sync_copy(r_hbm.at[src_v.at[j]], bufs[b])
                pltpu.async_copy(bufs[b], agg_sh.at[dst_v.at[j]], sems[b],
                                 add=True)
            return 0
        lax.fori_loop(0, HCH // 2, body, 0)
        for b in range(2):
            pltpu.make_async_copy(bufs[b], agg_sh.at[dst_v.at[0]],
                                  sems[b]).wait()
    plsc.subcore_barrier()

    pltpu.sync_copy(agg_sh.at[pl.ds(sid * NROWS, NROWS)],
                    out_hbm.at[cid, pl.ds(sid * NROWS, NROWS)])


_edge_agg = functools.partial(
    pl.kernel,
    out_type=jax.ShapeDtypeStruct((NC, NPAD, EMB), F32),
    mesh=_mesh,
    scratch_types=[
        pltpu.VMEM((HCH, CH), I32),
        pltpu.VMEM((HCH, CH), I32),
        pltpu.VMEM((CH, EMB), F32),
        pltpu.VMEM((CH, EMB), F32),
        pltpu.VMEM_SHARED((NPAD, EMB), F32),
        pltpu.SemaphoreType.DMA,
        pltpu.SemaphoreType.DMA,
    ],
)(_edge_body)


def _seg_body(h_hbm, bat_hbm, out_hbm, bat_v, buf, tab_sh):
    cid = lax.axis_index("c")
    sid = lax.axis_index("s")
    wid = cid * NS + sid

    pltpu.sync_copy(bat_hbm.at[wid], bat_v)

    _zero_buf(buf, GROWS)
    pltpu.sync_copy(buf.at[pl.ds(0, GROWS)],
                    tab_sh.at[pl.ds(sid * GROWS, GROWS)])
    plsc.subcore_barrier()

    def chunk(j, _):
        pltpu.sync_copy(h_hbm.at[pl.ds(wid * RPT + j * SCH, SCH)], buf)
        pltpu.sync_copy(buf, tab_sh.at[bat_v.at[j]], add=True)
        return 0
    lax.fori_loop(0, NSCH, chunk, 0)
    plsc.subcore_barrier()

    pltpu.sync_copy(tab_sh.at[pl.ds(sid * GROWS, GROWS)],
                    out_hbm.at[cid, pl.ds(sid * GROWS, GROWS)])


_seg_sum = functools.partial(
    pl.kernel,
    out_type=jax.ShapeDtypeStruct((NC, GPAD, EMB), F32),
    mesh=_mesh,
    scratch_types=[
        pltpu.VMEM((NSCH, SCH), I32),
        pltpu.VMEM((SCH, EMB), F32),
        pltpu.VMEM_SHARED((GPAD, EMB), F32),
    ],
)(_seg_body)


def _vnb_body(vn_hbm, bat_hbm, out_hbm, bat_v, buf):
    cid = lax.axis_index("c")
    sid = lax.axis_index("s")
    wid = cid * NS + sid

    pltpu.sync_copy(bat_hbm.at[wid], bat_v)

    def chunk(j, _):
        pltpu.sync_copy(vn_hbm.at[bat_v.at[j]], buf)
        pltpu.sync_copy(buf, out_hbm.at[pl.ds(wid * RPT + j * SCH, SCH)])
        return 0
    lax.fori_loop(0, NSCH, chunk, 0)


_vnb_gather = functools.partial(
    pl.kernel,
    out_type=jax.ShapeDtypeStruct((NPAD, EMB), F32),
    mesh=_mesh,
    scratch_types=[
        pltpu.VMEM((NSCH, SCH), I32),
        pltpu.VMEM((SCH, EMB), F32),
    ],
)(_vnb_body)


_RB = 2048


def _proj_body(x_ref, w_ref, b_ref, h_ref, r_ref):
    h = jnp.dot(x_ref[...], w_ref[...], preferred_element_type=F32) + b_ref[...]
    h_ref[...] = h
    r_ref[...] = jnp.maximum(h, 0.0)


_proj = pl.pallas_call(
    _proj_body,
    grid=(NPAD // _RB,),
    in_specs=[
        pl.BlockSpec((_RB, EMB), lambda i: (i, 0)),
        pl.BlockSpec((EMB, EMB), lambda i: (0, 0)),
        pl.BlockSpec((1, EMB), lambda i: (0, 0)),
    ],
    out_specs=[pl.BlockSpec((_RB, EMB), lambda i: (i, 0))] * 2,
    out_shape=[jax.ShapeDtypeStruct((NPAD, EMB), F32)] * 2,
)


def _pre_body(h_ref, vnb_ref, hp_ref, r_ref):
    hp = h_ref[...] + vnb_ref[...]
    hp_ref[...] = hp
    r_ref[...] = jnp.maximum(hp, 0.0)


_pre = pl.pallas_call(
    _pre_body,
    grid=(NPAD // _RB,),
    in_specs=[pl.BlockSpec((_RB, EMB), lambda i: (i, 0))] * 2,
    out_specs=[pl.BlockSpec((_RB, EMB), lambda i: (i, 0))] * 2,
    out_shape=[jax.ShapeDtypeStruct((NPAD, EMB), F32)] * 2,
)


def _mlp_body(relu_out, hp_ref, a0_ref, a1_ref, epsb_ref, w1_ref, b1_ref,
              g1_ref, be1_ref, w2_ref, b2_ref, g_ref, b_ref, o_ref):
    y = hp_ref[...] * epsb_ref[...] + (a0_ref[...] + a1_ref[...])
    t = jnp.dot(y, w1_ref[...], preferred_element_type=F32) + b1_ref[...]
    t = jnp.maximum(t * g1_ref[...] + be1_ref[...], 0.0)
    t = jnp.dot(t, w2_ref[...], preferred_element_type=F32) + b2_ref[...]
    t = t * g_ref[...] + b_ref[...]
    if relu_out:
        t = jnp.maximum(t, 0.0)
    o_ref[...] = t


def _make_mlp(relu_out):
    return pl.pallas_call(
        functools.partial(_mlp_body, relu_out),
        grid=(NPAD // _RB,),
        in_specs=[
            pl.BlockSpec((_RB, EMB), lambda i: (i, 0)),
            pl.BlockSpec((_RB, EMB), lambda i: (i, 0)),
            pl.BlockSpec((_RB, EMB), lambda i: (i, 0)),
            pl.BlockSpec((1, EMB), lambda i: (0, 0)),
            pl.BlockSpec((EMB, 2 * EMB), lambda i: (0, 0)),
            pl.BlockSpec((1, 2 * EMB), lambda i: (0, 0)),
            pl.BlockSpec((1, 2 * EMB), lambda i: (0, 0)),
            pl.BlockSpec((1, 2 * EMB), lambda i: (0, 0)),
            pl.BlockSpec((2 * EMB, EMB), lambda i: (0, 0)),
            pl.BlockSpec((1, EMB), lambda i: (0, 0)),
            pl.BlockSpec((1, EMB), lambda i: (0, 0)),
            pl.BlockSpec((1, EMB), lambda i: (0, 0)),
        ],
        out_specs=pl.BlockSpec((_RB, EMB), lambda i: (i, 0)),
        out_shape=jax.ShapeDtypeStruct((NPAD, EMB), F32),
    )


_mlp_mid = _make_mlp(True)
_mlp_last = _make_mlp(False)


def _vnmlp_body(s0_ref, s1_ref, vn_ref, w1_ref, b1_ref, g1_ref, be1_ref,
                w2_ref, b2_ref, g2_ref, be2_ref, o_ref):
    vtmp = s0_ref[...] + s1_ref[...] + vn_ref[...]
    u = jnp.dot(vtmp, w1_ref[...], preferred_element_type=F32) + b1_ref[...]
    u = jnp.maximum(u * g1_ref[...] + be1_ref[...], 0.0)
    u = jnp.dot(u, w2_ref[...], preferred_element_type=F32) + b2_ref[...]
    u = u * g2_ref[...] + be2_ref[...]
    o_ref[...] = jnp.maximum(u, 0.0)


_vnmlp = pl.pallas_call(
    _vnmlp_body,
    grid=(1,),
    in_specs=[
        pl.BlockSpec((GPAD, EMB), lambda i: (0, 0)),
        pl.BlockSpec((GPAD, EMB), lambda i: (0, 0)),
        pl.BlockSpec((GPAD, EMB), lambda i: (0, 0)),
        pl.BlockSpec((EMB, 2 * EMB), lambda i: (0, 0)),
        pl.BlockSpec((1, 2 * EMB), lambda i: (0, 0)),
        pl.BlockSpec((1, 2 * EMB), lambda i: (0, 0)),
        pl.BlockSpec((1, 2 * EMB), lambda i: (0, 0)),
        pl.BlockSpec((2 * EMB, EMB), lambda i: (0, 0)),
        pl.BlockSpec((1, EMB), lambda i: (0, 0)),
        pl.BlockSpec((1, EMB), lambda i: (0, 0)),
        pl.BlockSpec((1, EMB), lambda i: (0, 0)),
    ],
    out_specs=pl.BlockSpec((GPAD, EMB), lambda i: (0, 0)),
    out_shape=jax.ShapeDtypeStruct((GPAD, EMB), F32),
)


def _head_body(s0_ref, s1_ref, bat_ref, g_ref, b_ref, wp_ref, bp_ref,
               out_ref, ge_ref):
    sums = s0_ref[...] + s1_ref[...]
    bat = bat_ref[...]
    gid = lax.broadcasted_iota(I32, (NUM_GRAPHS, 1), 0)

    cnt = jnp.zeros((NUM_GRAPHS, 1), F32)
    for i in range(NPAD // EMB):
        eq = (bat[i:i + 1, :] == gid).astype(F32)
        cnt = cnt + jnp.sum(eq, axis=1, keepdims=True)
    ge = sums / jnp.maximum(cnt, 1.0)
    mu = jnp.mean(ge, axis=1, keepdims=True)
    var = jnp.mean((ge - mu) ** 2, axis=1, keepdims=True)
    ge = (ge - mu) / jnp.sqrt(var + 1e-5) * g_ref[...] + b_ref[...]
    ge_ref[...] = ge
    out_ref[...] = (jnp.dot(ge, wp_ref[...], preferred_element_type=F32)
                    + bp_ref[...])


_head = pl.pallas_call(
    _head_body,
    grid=(1,),
    in_specs=[
        pl.BlockSpec((NUM_GRAPHS, EMB), lambda i: (0, 0)),
        pl.BlockSpec((NUM_GRAPHS, EMB), lambda i: (0, 0)),
        pl.BlockSpec((NPAD // EMB, EMB), lambda i: (0, 0)),
        pl.BlockSpec((1, EMB), lambda i: (0, 0)),
        pl.BlockSpec((1, EMB), lambda i: (0, 0)),
        pl.BlockSpec((EMB, NUM_CLASS), lambda i: (0, 0)),
        pl.BlockSpec((1, NUM_CLASS), lambda i: (0, 0)),
    ],
    out_specs=[pl.BlockSpec((NUM_GRAPHS, NUM_CLASS), lambda i: (0, 0)),
               pl.BlockSpec((NUM_GRAPHS, EMB), lambda i: (0, 0))],
    out_shape=[jax.ShapeDtypeStruct((NUM_GRAPHS, NUM_CLASS), F32),
               jax.ShapeDtypeStruct((NUM_GRAPHS, EMB), F32)],
)


def _row(v):
    return v.reshape(1, -1).astype(F32)


def kernel(x, edge_index, batch, params):
    xp = jnp.pad(x, ((0, NPAD - N_NODES), (0, 0)))
    batp = jnp.pad(batch.astype(I32), (0, NPAD - N_NODES),
                   constant_values=NUM_GRAPHS)
    bat_slab = batp.reshape(NW, NSCH, SCH)
    src = jnp.pad(edge_index[0].astype(I32), (0, EPAD - N_EDGES))
    dst = jnp.pad(edge_index[1].astype(I32), (0, EPAD - N_EDGES),
                  constant_values=JUNK_ROW)
    src_slab = src.reshape(NW, NCHUNK, CH)
    dst_slab = dst.reshape(NW, NCHUNK, CH)

    h, r = _proj(xp, params['Win'], _row(params['bin']))
    vn = jnp.zeros((GPAD, EMB), F32)

    for l in range(NUM_LAYER):
        if l > 0:
            vnb = _vnb_gather(vn, bat_slab)
            hp, r = _pre(h, vnb)
        else:
            hp = h
        p = params['gin%d' % l]
        q = params['bn%d' % l]
        agg = _edge_agg(r, src_slab, dst_slab)
        epsb = (1.0 + p['eps']) * jnp.ones((1, EMB), F32)
        mlp = _mlp_mid if l < NUM_LAYER - 1 else _mlp_last
        h_next = mlp(hp, agg[0], agg[1], epsb, p['W1'], _row(p['b1']),
                     _row(p['g1']), _row(p['be1']), p['W2'], _row(p['b2']),
                     _row(q['g']), _row(q['b']))
        if l < NUM_LAYER - 1:
            st = _seg_sum(h, bat_slab)
            v = params['vn%d' % l]
            vn = _vnmlp(st[0], st[1], vn, v['W1'], _row(v['b1']),
                        _row(v['g1']), _row(v['be1']), v['W2'], _row(v['b2']),
                        _row(v['g2']), _row(v['be2']))
        h = h_next

    st = _seg_sum(h, bat_slab)
    out, ge = _head(st[0], st[1], batp.reshape(NPAD // EMB, EMB),
                    _row(params['ln']['g']), _row(params['ln']['b']),
                    params['Wp'], _row(params['bp']))
    return out, ge


# speedup vs baseline: 1.8263x; 1.8263x over previous
import functools

import jax
import jax.numpy as jnp
from jax import lax
from jax.experimental import pallas as pl
from jax.experimental.pallas import tpu as pltpu
from jax.experimental.pallas import tpu_sc as plsc

F32 = jnp.float32
I32 = jnp.int32

N_NODES = 10000
N_EDGES = 320000
EMB = 128
NUM_CLASS = 128
NUM_LAYER = 5
NUM_GRAPHS = 512

NC, NS = 2, 16
NW = NC * NS
NPAD = 10240
GPAD = 640
JUNK_ROW = NPAD - 1

CH = 128
FAST_CID = 0
NCHF = 102
NCHS = 55
EPAD_F = NS * NCHF * CH
EPAD_S = NS * NCHS * CH
EPAD = EPAD_F + EPAD_S

RPT = NPAD // NW
SCH = 64
NSCH = RPT // SCH
NROWS = NPAD // NS
GROWS = GPAD // NS

_mesh = plsc.VectorSubcoreMesh(core_axis_name="c", subcore_axis_name="s")


def _zero_buf(buf, nrows):
    def body(i, _):
        buf[i // 8, pl.ds((i % 8) * 16, 16)] = jnp.zeros((16,), F32)
        return 0
    lax.fori_loop(0, nrows * 8, body, 0)


def _edge_body(r_hbm, src_hbm, dst_hbm, out_hbm, src_v, dst_v, buf, agg_sh):
    cid = lax.axis_index("c")
    sid = lax.axis_index("s")
    wid = cid * NS + sid
    nch = jnp.where(cid == FAST_CID, NCHF, NCHS)

    pltpu.sync_copy(src_hbm.at[wid], src_v)
    pltpu.sync_copy(dst_hbm.at[wid], dst_v)

    _zero_buf(buf, CH)
    def zcopy(i, _):
        pltpu.sync_copy(buf, agg_sh.at[pl.ds(sid * NROWS + i * CH, CH)])
        return 0
    lax.fori_loop(0, NROWS // CH, zcopy, 0)
    plsc.subcore_barrier()

    def chunk(j, _):
        pltpu.sync_copy(r_hbm.at[src_v.at[j]], buf)
        pltpu.sync_copy(buf, agg_sh.at[dst_v.at[j]], add=True)
        return 0
    lax.fori_loop(0, nch, chunk, 0)
    plsc.subcore_barrier()

    pltpu.sync_copy(agg_sh.at[pl.ds(sid * NROWS, NROWS)],
                    out_hbm.at[cid, pl.ds(sid * NROWS, NROWS)])


_edge_agg = functools.partial(
    pl.kernel,
    out_type=jax.ShapeDtypeStruct((NC, NPAD, EMB), F32),
    mesh=_mesh,
    scratch_types=[
        pltpu.VMEM((NCHF, CH), I32),
        pltpu.VMEM((NCHF, CH), I32),
        pltpu.VMEM((CH, EMB), F32),
        pltpu.VMEM_SHARED((NPAD, EMB), F32),
    ],
)(_edge_body)


def _seg_body(h_hbm, bat_hbm, out_hbm, bat_v, buf, tab_sh):
    cid = lax.axis_index("c")
    sid = lax.axis_index("s")
    wid = cid * NS + sid

    pltpu.sync_copy(bat_hbm.at[wid], bat_v)

    _zero_buf(buf, GROWS)
    pltpu.sync_copy(buf.at[pl.ds(0, GROWS)],
                    tab_sh.at[pl.ds(sid * GROWS, GROWS)])
    plsc.subcore_barrier()

    def chunk(j, _):
        pltpu.sync_copy(h_hbm.at[pl.ds(wid * RPT + j * SCH, SCH)], buf)
        pltpu.sync_copy(buf, tab_sh.at[bat_v.at[j]], add=True)
        return 0
    lax.fori_loop(0, NSCH, chunk, 0)
    plsc.subcore_barrier()

    pltpu.sync_copy(tab_sh.at[pl.ds(sid * GROWS, GROWS)],
                    out_hbm.at[cid, pl.ds(sid * GROWS, GROWS)])


_seg_sum = functools.partial(
    pl.kernel,
    out_type=jax.ShapeDtypeStruct((NC, GPAD, EMB), F32),
    mesh=_mesh,
    scratch_types=[
        pltpu.VMEM((NSCH, SCH), I32),
        pltpu.VMEM((SCH, EMB), F32),
        pltpu.VMEM_SHARED((GPAD, EMB), F32),
    ],
)(_seg_body)


def _vnb_body(vn_hbm, bat_hbm, out_hbm, bat_v, buf):
    cid = lax.axis_index("c")
    sid = lax.axis_index("s")
    wid = cid * NS + sid

    pltpu.sync_copy(bat_hbm.at[wid], bat_v)

    def chunk(j, _):
        pltpu.sync_copy(vn_hbm.at[bat_v.at[j]], buf)
        pltpu.sync_copy(buf, out_hbm.at[pl.ds(wid * RPT + j * SCH, SCH)])
        return 0
    lax.fori_loop(0, NSCH, chunk, 0)


_vnb_gather = functools.partial(
    pl.kernel,
    out_type=jax.ShapeDtypeStruct((NPAD, EMB), F32),
    mesh=_mesh,
    scratch_types=[
        pltpu.VMEM((NSCH, SCH), I32),
        pltpu.VMEM((SCH, EMB), F32),
    ],
)(_vnb_body)


_RB = 2048


def _proj_body(x_ref, w_ref, b_ref, h_ref, r_ref):
    h = jnp.dot(x_ref[...], w_ref[...], preferred_element_type=F32) + b_ref[...]
    h_ref[...] = h
    r_ref[...] = jnp.maximum(h, 0.0)


_proj = pl.pallas_call(
    _proj_body,
    grid=(NPAD // _RB,),
    in_specs=[
        pl.BlockSpec((_RB, EMB), lambda i: (i, 0)),
        pl.BlockSpec((EMB, EMB), lambda i: (0, 0)),
        pl.BlockSpec((1, EMB), lambda i: (0, 0)),
    ],
    out_specs=[pl.BlockSpec((_RB, EMB), lambda i: (i, 0))] * 2,
    out_shape=[jax.ShapeDtypeStruct((NPAD, EMB), F32)] * 2,
)


def _pre_body(h_ref, vnb_ref, hp_ref, r_ref):
    hp = h_ref[...] + vnb_ref[...]
    hp_ref[...] = hp
    r_ref[...] = jnp.maximum(hp, 0.0)


_pre = pl.pallas_call(
    _pre_body,
    grid=(NPAD // _RB,),
    in_specs=[pl.BlockSpec((_RB, EMB), lambda i: (i, 0))] * 2,
    out_specs=[pl.BlockSpec((_RB, EMB), lambda i: (i, 0))] * 2,
    out_shape=[jax.ShapeDtypeStruct((NPAD, EMB), F32)] * 2,
)


def _mlp_body(relu_out, hp_ref, a0_ref, a1_ref, epsb_ref, w1_ref, b1_ref,
              g1_ref, be1_ref, w2_ref, b2_ref, g_ref, b_ref, o_ref):
    y = hp_ref[...] * epsb_ref[...] + (a0_ref[...] + a1_ref[...])
    t = jnp.dot(y, w1_ref[...], preferred_element_type=F32) + b1_ref[...]
    t = jnp.maximum(t * g1_ref[...] + be1_ref[...], 0.0)
    t = jnp.dot(t, w2_ref[...], preferred_element_type=F32) + b2_ref[...]
    t = t * g_ref[...] + b_ref[...]
    if relu_out:
        t = jnp.maximum(t, 0.0)
    o_ref[...] = t


def _make_mlp(relu_out):
    return pl.pallas_call(
        functools.partial(_mlp_body, relu_out),
        grid=(NPAD // _RB,),
        in_specs=[
            pl.BlockSpec((_RB, EMB), lambda i: (i, 0)),
            pl.BlockSpec((_RB, EMB), lambda i: (i, 0)),
            pl.BlockSpec((_RB, EMB), lambda i: (i, 0)),
            pl.BlockSpec((1, EMB), lambda i: (0, 0)),
            pl.BlockSpec((EMB, 2 * EMB), lambda i: (0, 0)),
            pl.BlockSpec((1, 2 * EMB), lambda i: (0, 0)),
            pl.BlockSpec((1, 2 * EMB), lambda i: (0, 0)),
            pl.BlockSpec((1, 2 * EMB), lambda i: (0, 0)),
            pl.BlockSpec((2 * EMB, EMB), lambda i: (0, 0)),
            pl.BlockSpec((1, EMB), lambda i: (0, 0)),
            pl.BlockSpec((1, EMB), lambda i: (0, 0)),
            pl.BlockSpec((1, EMB), lambda i: (0, 0)),
        ],
        out_specs=pl.BlockSpec((_RB, EMB), lambda i: (i, 0)),
        out_shape=jax.ShapeDtypeStruct((NPAD, EMB), F32),
    )


_mlp_mid = _make_mlp(True)
_mlp_last = _make_mlp(False)


def _vnmlp_body(s0_ref, s1_ref, vn_ref, w1_ref, b1_ref, g1_ref, be1_ref,
                w2_ref, b2_ref, g2_ref, be2_ref, o_ref):
    vtmp = s0_ref[...] + s1_ref[...] + vn_ref[...]
    u = jnp.dot(vtmp, w1_ref[...], preferred_element_type=F32) + b1_ref[...]
    u = jnp.maximum(u * g1_ref[...] + be1_ref[...], 0.0)
    u = jnp.dot(u, w2_ref[...], preferred_element_type=F32) + b2_ref[...]
    u = u * g2_ref[...] + be2_ref[...]
    o_ref[...] = jnp.maximum(u, 0.0)


_vnmlp = pl.pallas_call(
    _vnmlp_body,
    grid=(1,),
    in_specs=[
        pl.BlockSpec((GPAD, EMB), lambda i: (0, 0)),
        pl.BlockSpec((GPAD, EMB), lambda i: (0, 0)),
        pl.BlockSpec((GPAD, EMB), lambda i: (0, 0)),
        pl.BlockSpec((EMB, 2 * EMB), lambda i: (0, 0)),
        pl.BlockSpec((1, 2 * EMB), lambda i: (0, 0)),
        pl.BlockSpec((1, 2 * EMB), lambda i: (0, 0)),
        pl.BlockSpec((1, 2 * EMB), lambda i: (0, 0)),
        pl.BlockSpec((2 * EMB, EMB), lambda i: (0, 0)),
        pl.BlockSpec((1, EMB), lambda i: (0, 0)),
        pl.BlockSpec((1, EMB), lambda i: (0, 0)),
        pl.BlockSpec((1, EMB), lambda i: (0, 0)),
    ],
    out_specs=pl.BlockSpec((GPAD, EMB), lambda i: (0, 0)),
    out_shape=jax.ShapeDtypeStruct((GPAD, EMB), F32),
)


def _head_body(s0_ref, s1_ref, bat_ref, g_ref, b_ref, wp_ref, bp_ref,
               out_ref, ge_ref):
    sums = s0_ref[...] + s1_ref[...]
    bat = bat_ref[...]
    gid = lax.broadcasted_iota(I32, (NUM_GRAPHS, 1), 0)

    cnt = jnp.zeros((NUM_GRAPHS, 1), F32)
    for i in range(NPAD // EMB):
        eq = (bat[i:i + 1, :] == gid).astype(F32)
        cnt = cnt + jnp.sum(eq, axis=1, keepdims=True)
    ge = sums / jnp.maximum(cnt, 1.0)
    mu = jnp.mean(ge, axis=1, keepdims=True)
    var = jnp.mean((ge - mu) ** 2, axis=1, keepdims=True)
    ge = (ge - mu) / jnp.sqrt(var + 1e-5) * g_ref[...] + b_ref[...]
    ge_ref[...] = ge
    out_ref[...] = (jnp.dot(ge, wp_ref[...], preferred_element_type=F32)
                    + bp_ref[...])


_head = pl.pallas_call(
    _head_body,
    grid=(1,),
    in_specs=[
        pl.BlockSpec((NUM_GRAPHS, EMB), lambda i: (0, 0)),
        pl.BlockSpec((NUM_GRAPHS, EMB), lambda i: (0, 0)),
        pl.BlockSpec((NPAD // EMB, EMB), lambda i: (0, 0)),
        pl.BlockSpec((1, EMB), lambda i: (0, 0)),
        pl.BlockSpec((1, EMB), lambda i: (0, 0)),
        pl.BlockSpec((EMB, NUM_CLASS), lambda i: (0, 0)),
        pl.BlockSpec((1, NUM_CLASS), lambda i: (0, 0)),
    ],
    out_specs=[pl.BlockSpec((NUM_GRAPHS, NUM_CLASS), lambda i: (0, 0)),
               pl.BlockSpec((NUM_GRAPHS, EMB), lambda i: (0, 0))],
    out_shape=[jax.ShapeDtypeStruct((NUM_GRAPHS, NUM_CLASS), F32),
               jax.ShapeDtypeStruct((NUM_GRAPHS, EMB), F32)],
)


def _row(v):
    return v.reshape(1, -1).astype(F32)


def kernel(x, edge_index, batch, params):
    xp = jnp.pad(x, ((0, NPAD - N_NODES), (0, 0)))
    batp = jnp.pad(batch.astype(I32), (0, NPAD - N_NODES),
                   constant_values=NUM_GRAPHS)
    bat_slab = batp.reshape(NW, NSCH, SCH)
    def _slab(a, padval):
        ap = jnp.pad(a.astype(I32), (0, EPAD - N_EDGES),
                     constant_values=padval)
        fa = ap[:EPAD_F].reshape(NS, NCHF, CH)
        sa = jnp.pad(ap[EPAD_F:].reshape(NS, NCHS, CH),
                     ((0, 0), (0, NCHF - NCHS), (0, 0)),
                     constant_values=padval)
        parts = (fa, sa) if FAST_CID == 0 else (sa, fa)
        return jnp.concatenate(parts, axis=0)

    src_slab = _slab(edge_index[0], 0)
    dst_slab = _slab(edge_index[1], JUNK_ROW)

    h, r = _proj(xp, params['Win'], _row(params['bin']))
    vn = jnp.zeros((GPAD, EMB), F32)

    for l in range(NUM_LAYER):
        if l > 0:
            vnb = _vnb_gather(vn, bat_slab)
            hp, r = _pre(h, vnb)
        else:
            hp = h
        p = params['gin%d' % l]
        q = params['bn%d' % l]
        agg = _edge_agg(r, src_slab, dst_slab)
        epsb = (1.0 + p['eps']) * jnp.ones((1, EMB), F32)
        mlp = _mlp_mid if l < NUM_LAYER - 1 else _mlp_last
        h_next = mlp(hp, agg[0], agg[1], epsb, p['W1'], _row(p['b1']),
                     _row(p['g1']), _row(p['be1']), p['W2'], _row(p['b2']),
                     _row(q['g']), _row(q['b']))
        if l < NUM_LAYER - 1:
            st = _seg_sum(h, bat_slab)
            v = params['vn%d' % l]
            vn = _vnmlp(st[0], st[1], vn, v['W1'], _row(v['b1']),
                        _row(v['g1']), _row(v['be1']), v['W2'], _row(v['b2']),
                        _row(v['g2']), _row(v['be2']))
        h = h_next

    st = _seg_sum(h, bat_slab)
    out, ge = _head(st[0], st[1], batp.reshape(NPAD // EMB, EMB),
                    _row(params['ln']['g']), _row(params['ln']['b']),
                    params['Wp'], _row(params['bp']))
    return out, ge


# speedup vs baseline: 1.9628x; 1.0747x over previous
import functools

import jax
import jax.numpy as jnp
from jax import lax
from jax.experimental import pallas as pl
from jax.experimental.pallas import tpu as pltpu
from jax.experimental.pallas import tpu_sc as plsc

F32 = jnp.float32
I32 = jnp.int32

N_NODES = 10000
N_EDGES = 320000
EMB = 128
NUM_CLASS = 128
NUM_LAYER = 5
NUM_GRAPHS = 512

NC, NS = 2, 16
NW = NC * NS
NPAD = 10240
GPAD = 640
JUNK_ROW = NPAD - 1

CH = 128
FAST_CID = 0
NCHF = 102
NCHS = 55
EPAD_F = NS * NCHF * CH
EPAD_S = NS * NCHS * CH
EPAD = EPAD_F + EPAD_S

NROWS = NPAD // NS

_mesh = plsc.VectorSubcoreMesh(core_axis_name="c", subcore_axis_name="s")


def _zero_buf(buf, nrows):
    def body(i, _):
        buf[i // 8, pl.ds((i % 8) * 16, 16)] = jnp.zeros((16,), F32)
        return 0
    lax.fori_loop(0, nrows * 8, body, 0)


def _edge_body(r_hbm, src_hbm, dst_hbm, out_hbm, src_v, dst_v, buf, agg_sh):
    cid = lax.axis_index("c")
    sid = lax.axis_index("s")
    wid = cid * NS + sid
    nch = jnp.where(cid == FAST_CID, NCHF, NCHS)

    pltpu.sync_copy(src_hbm.at[wid], src_v)
    pltpu.sync_copy(dst_hbm.at[wid], dst_v)

    _zero_buf(buf, CH)
    def zcopy(i, _):
        pltpu.sync_copy(buf, agg_sh.at[pl.ds(sid * NROWS + i * CH, CH)])
        return 0
    lax.fori_loop(0, NROWS // CH, zcopy, 0)
    plsc.subcore_barrier()

    def chunk(j, _):
        pltpu.sync_copy(r_hbm.at[src_v.at[j]], buf)
        pltpu.sync_copy(buf, agg_sh.at[dst_v.at[j]], add=True)
        return 0
    lax.fori_loop(0, nch, chunk, 0)
    plsc.subcore_barrier()

    pltpu.sync_copy(agg_sh.at[pl.ds(sid * NROWS, NROWS)],
                    out_hbm.at[cid, pl.ds(sid * NROWS, NROWS)])


_edge_agg = functools.partial(
    pl.kernel,
    out_type=jax.ShapeDtypeStruct((NC, NPAD, EMB), F32),
    mesh=_mesh,
    scratch_types=[
        pltpu.VMEM((NCHF, CH), I32),
        pltpu.VMEM((NCHF, CH), I32),
        pltpu.VMEM((CH, EMB), F32),
        pltpu.VMEM_SHARED((NPAD, EMB), F32),
    ],
)(_edge_body)


_RB = 2048
NB = NPAD // _RB


def _proj_body(x_ref, w_ref, b_ref, h_ref, r_ref):
    h = jnp.dot(x_ref[...], w_ref[...], preferred_element_type=F32) + b_ref[...]
    h_ref[...] = h
    r_ref[...] = jnp.maximum(h, 0.0)


_proj = pl.pallas_call(
    _proj_body,
    grid=(NB,),
    in_specs=[
        pl.BlockSpec((_RB, EMB), lambda i: (i, 0)),
        pl.BlockSpec((EMB, EMB), lambda i: (0, 0)),
        pl.BlockSpec((1, EMB), lambda i: (0, 0)),
    ],
    out_specs=[pl.BlockSpec((_RB, EMB), lambda i: (i, 0))] * 2,
    out_shape=[jax.ShapeDtypeStruct((NPAD, EMB), F32)] * 2,
)


def _pre_body(h_ref, batc_ref, vn_ref, hp_ref, r_ref):
    oh = (batc_ref[...] == lax.broadcasted_iota(I32, (1, GPAD), 1)).astype(F32)
    hp = h_ref[...] + jnp.dot(oh, vn_ref[...], preferred_element_type=F32)
    hp_ref[...] = hp
    r_ref[...] = jnp.maximum(hp, 0.0)


_pre = pl.pallas_call(
    _pre_body,
    grid=(NB,),
    in_specs=[
        pl.BlockSpec((_RB, EMB), lambda i: (i, 0)),
        pl.BlockSpec((_RB, 1), lambda i: (i, 0)),
        pl.BlockSpec((GPAD, EMB), lambda i: (0, 0)),
    ],
    out_specs=[pl.BlockSpec((_RB, EMB), lambda i: (i, 0))] * 2,
    out_shape=[jax.ShapeDtypeStruct((NPAD, EMB), F32)] * 2,
)


def _mlp_body(relu_out, hp_ref, a0_ref, a1_ref, epsb_ref, w1_ref, b1_ref,
              g1_ref, be1_ref, w2_ref, b2_ref, g_ref, b_ref, o_ref):
    y = hp_ref[...] * epsb_ref[...] + (a0_ref[...] + a1_ref[...])
    t = jnp.dot(y, w1_ref[...], preferred_element_type=F32) + b1_ref[...]
    t = jnp.maximum(t * g1_ref[...] + be1_ref[...], 0.0)
    t = jnp.dot(t, w2_ref[...], preferred_element_type=F32) + b2_ref[...]
    t = t * g_ref[...] + b_ref[...]
    if relu_out:
        t = jnp.maximum(t, 0.0)
    o_ref[...] = t


def _make_mlp(relu_out):
    return pl.pallas_call(
        functools.partial(_mlp_body, relu_out),
        grid=(NB,),
        in_specs=[
            pl.BlockSpec((_RB, EMB), lambda i: (i, 0)),
            pl.BlockSpec((_RB, EMB), lambda i: (i, 0)),
            pl.BlockSpec((_RB, EMB), lambda i: (i, 0)),
            pl.BlockSpec((1, EMB), lambda i: (0, 0)),
            pl.BlockSpec((EMB, 2 * EMB), lambda i: (0, 0)),
            pl.BlockSpec((1, 2 * EMB), lambda i: (0, 0)),
            pl.BlockSpec((1, 2 * EMB), lambda i: (0, 0)),
            pl.BlockSpec((1, 2 * EMB), lambda i: (0, 0)),
            pl.BlockSpec((2 * EMB, EMB), lambda i: (0, 0)),
            pl.BlockSpec((1, EMB), lambda i: (0, 0)),
            pl.BlockSpec((1, EMB), lambda i: (0, 0)),
            pl.BlockSpec((1, EMB), lambda i: (0, 0)),
        ],
        out_specs=pl.BlockSpec((_RB, EMB), lambda i: (i, 0)),
        out_shape=jax.ShapeDtypeStruct((NPAD, EMB), F32),
    )


_mlp_mid = _make_mlp(True)
_mlp_last = _make_mlp(False)


_DN0 = (((0,), (0,)), ((), ()))


def _vnmlp_body(hp_ref, batc_ref, vn_ref, w1_ref, b1_ref, g1_ref, be1_ref,
                w2_ref, b2_ref, g2_ref, be2_ref, o_ref, acc_ref, cnt_ref):
    i = pl.program_id(0)

    @pl.when(i == 0)
    def _():
        acc_ref[...] = jnp.zeros_like(acc_ref)
        cnt_ref[...] = jnp.zeros_like(cnt_ref)

    oh = (batc_ref[...] == lax.broadcasted_iota(I32, (1, GPAD), 1)).astype(F32)
    acc_ref[...] += lax.dot_general(oh, hp_ref[...], _DN0,
                                    preferred_element_type=F32)
    cnt_ref[...] += lax.dot_general(oh, jnp.ones((_RB, 1), F32), _DN0,
                                    preferred_element_type=F32)

    @pl.when(i == NB - 1)
    def _():
        vn = vn_ref[...]
        vtmp = acc_ref[...] - cnt_ref[...] * vn + vn
        u = jnp.dot(vtmp, w1_ref[...], preferred_element_type=F32) + b1_ref[...]
        u = jnp.maximum(u * g1_ref[...] + be1_ref[...], 0.0)
        u = jnp.dot(u, w2_ref[...], preferred_element_type=F32) + b2_ref[...]
        u = u * g2_ref[...] + be2_ref[...]
        o_ref[...] = jnp.maximum(u, 0.0)


_vnmlp = pl.pallas_call(
    _vnmlp_body,
    grid=(NB,),
    in_specs=[
        pl.BlockSpec((_RB, EMB), lambda i: (i, 0)),
        pl.BlockSpec((_RB, 1), lambda i: (i, 0)),
        pl.BlockSpec((GPAD, EMB), lambda i: (0, 0)),
        pl.BlockSpec((EMB, 2 * EMB), lambda i: (0, 0)),
        pl.BlockSpec((1, 2 * EMB), lambda i: (0, 0)),
        pl.BlockSpec((1, 2 * EMB), lambda i: (0, 0)),
        pl.BlockSpec((1, 2 * EMB), lambda i: (0, 0)),
        pl.BlockSpec((2 * EMB, EMB), lambda i: (0, 0)),
        pl.BlockSpec((1, EMB), lambda i: (0, 0)),
        pl.BlockSpec((1, EMB), lambda i: (0, 0)),
        pl.BlockSpec((1, EMB), lambda i: (0, 0)),
    ],
    out_specs=pl.BlockSpec((GPAD, EMB), lambda i: (0, 0)),
    out_shape=jax.ShapeDtypeStruct((GPAD, EMB), F32),
    scratch_shapes=[
        pltpu.VMEM((GPAD, EMB), F32),
        pltpu.VMEM((GPAD, 1), F32),
    ],
)


def _head_body(h_ref, batc_ref, g_ref, b_ref, wp_ref, bp_ref,
               out_ref, ge_ref, acc_ref, cnt_ref):
    i = pl.program_id(0)

    @pl.when(i == 0)
    def _():
        acc_ref[...] = jnp.zeros_like(acc_ref)
        cnt_ref[...] = jnp.zeros_like(cnt_ref)

    oh = (batc_ref[...] ==
          lax.broadcasted_iota(I32, (1, NUM_GRAPHS), 1)).astype(F32)
    acc_ref[...] += lax.dot_general(oh, h_ref[...], _DN0,
                                    preferred_element_type=F32)
    cnt_ref[...] += lax.dot_general(oh, jnp.ones((_RB, 1), F32), _DN0,
                                    preferred_element_type=F32)

    @pl.when(i == NB - 1)
    def _():
        ge = acc_ref[...] / jnp.maximum(cnt_ref[...], 1.0)
        mu = jnp.mean(ge, axis=1, keepdims=True)
        var = jnp.mean((ge - mu) ** 2, axis=1, keepdims=True)
        ge = (ge - mu) / jnp.sqrt(var + 1e-5) * g_ref[...] + b_ref[...]
        ge_ref[...] = ge
        out_ref[...] = (jnp.dot(ge, wp_ref[...], preferred_element_type=F32)
                        + bp_ref[...])


_head = pl.pallas_call(
    _head_body,
    grid=(NB,),
    in_specs=[
        pl.BlockSpec((_RB, EMB), lambda i: (i, 0)),
        pl.BlockSpec((_RB, 1), lambda i: (i, 0)),
        pl.BlockSpec((1, EMB), lambda i: (0, 0)),
        pl.BlockSpec((1, EMB), lambda i: (0, 0)),
        pl.BlockSpec((EMB, NUM_CLASS), lambda i: (0, 0)),
        pl.BlockSpec((1, NUM_CLASS), lambda i: (0, 0)),
    ],
    out_specs=[pl.BlockSpec((NUM_GRAPHS, NUM_CLASS), lambda i: (0, 0)),
               pl.BlockSpec((NUM_GRAPHS, EMB), lambda i: (0, 0))],
    out_shape=[jax.ShapeDtypeStruct((NUM_GRAPHS, NUM_CLASS), F32),
               jax.ShapeDtypeStruct((NUM_GRAPHS, EMB), F32)],
    scratch_shapes=[
        pltpu.VMEM((NUM_GRAPHS, EMB), F32),
        pltpu.VMEM((NUM_GRAPHS, 1), F32),
    ],
)


def _row(v):
    return v.reshape(1, -1).astype(F32)


def kernel(x, edge_index, batch, params):
    xp = jnp.pad(x, ((0, NPAD - N_NODES), (0, 0)))
    batp = jnp.pad(batch.astype(I32), (0, NPAD - N_NODES),
                   constant_values=NUM_GRAPHS)
    batc = batp.reshape(NPAD, 1)

    def _slab(a, padval):
        ap = jnp.pad(a.astype(I32), (0, EPAD - N_EDGES),
                     constant_values=padval)
        fa = ap[:EPAD_F].reshape(NS, NCHF, CH)
        sa = jnp.pad(ap[EPAD_F:].reshape(NS, NCHS, CH),
                     ((0, 0), (0, NCHF - NCHS), (0, 0)),
                     constant_values=padval)
        parts = (fa, sa) if FAST_CID == 0 else (sa, fa)
        return jnp.concatenate(parts, axis=0)

    src_slab = _slab(edge_index[0], 0)
    dst_slab = _slab(edge_index[1], JUNK_ROW)

    h, r = _proj(xp, params['Win'], _row(params['bin']))
    hp = h
    vn = jnp.zeros((GPAD, EMB), F32)

    for l in range(NUM_LAYER):
        if l > 0:
            hp, r = _pre(h, batc, vn)
        p = params['gin%d' % l]
        q = params['bn%d' % l]
        agg = _edge_agg(r, src_slab, dst_slab)
        epsb = (1.0 + p['eps']) * jnp.ones((1, EMB), F32)
        mlp = _mlp_mid if l < NUM_LAYER - 1 else _mlp_last
        h_next = mlp(hp, agg[0], agg[1], epsb, p['W1'], _row(p['b1']),
                     _row(p['g1']), _row(p['be1']), p['W2'], _row(p['b2']),
                     _row(q['g']), _row(q['b']))
        if l < NUM_LAYER - 1:
            v = params['vn%d' % l]
            vn = _vnmlp(hp, batc, vn, v['W1'], _row(v['b1']),
                        _row(v['g1']), _row(v['be1']), v['W2'], _row(v['b2']),
                        _row(v['g2']), _row(v['be2']))
        h = h_next

    out, ge = _head(h, batc, _row(params['ln']['g']), _row(params['ln']['b']),
                    params['Wp'], _row(params['bp']))
    return out, ge


# speedup vs baseline: 2.1131x; 1.0766x over previous
import functools

import jax
import jax.numpy as jnp
from jax import lax
from jax.experimental import pallas as pl
from jax.experimental.pallas import tpu as pltpu
from jax.experimental.pallas import tpu_sc as plsc

F32 = jnp.float32
I32 = jnp.int32

N_NODES = 10000
N_EDGES = 320000
EMB = 128
NUM_CLASS = 128
NUM_LAYER = 5
NUM_GRAPHS = 512

NC, NS = 2, 16
NW = NC * NS
NPAD = 10240
GPAD = 640
JUNK_ROW = NPAD - 1

CH = 128
FAST_CID = 0
NCHF = 96
NCHS = 61
EPAD_F = NS * NCHF * CH
EPAD_S = NS * NCHS * CH
EPAD = EPAD_F + EPAD_S

NROWS = NPAD // NS

_mesh = plsc.VectorSubcoreMesh(core_axis_name="c", subcore_axis_name="s")


def _zero_buf(buf, nrows):
    def body(i, _):
        buf[i // 8, pl.ds((i % 8) * 16, 16)] = jnp.zeros((16,), F32)
        return 0
    lax.fori_loop(0, nrows * 8, body, 0)


def _edge_body(r_hbm, src_hbm, dst_hbm, out_hbm, src_v, dst_v, buf, agg_sh):
    cid = lax.axis_index("c")
    sid = lax.axis_index("s")
    wid = cid * NS + sid
    nch = jnp.where(cid == FAST_CID, NCHF, NCHS)

    pltpu.sync_copy(src_hbm.at[wid], src_v)
    pltpu.sync_copy(dst_hbm.at[wid], dst_v)

    _zero_buf(buf, CH)
    def zcopy(i, _):
        pltpu.sync_copy(buf, agg_sh.at[pl.ds(sid * NROWS + i * CH, CH)])
        return 0
    lax.fori_loop(0, NROWS // CH, zcopy, 0)
    plsc.subcore_barrier()

    def chunk(j, _):
        pltpu.sync_copy(r_hbm.at[src_v.at[j]], buf)
        pltpu.sync_copy(buf, agg_sh.at[dst_v.at[j]], add=True)
        return 0
    lax.fori_loop(0, nch, chunk, 0)
    plsc.subcore_barrier()

    pltpu.sync_copy(agg_sh.at[pl.ds(sid * NROWS, NROWS)],
                    out_hbm.at[cid, pl.ds(sid * NROWS, NROWS)])


_edge_agg = functools.partial(
    pl.kernel,
    out_type=jax.ShapeDtypeStruct((NC, NPAD, EMB), F32),
    mesh=_mesh,
    scratch_types=[
        pltpu.VMEM((NCHF, CH), I32),
        pltpu.VMEM((NCHF, CH), I32),
        pltpu.VMEM((CH, EMB), F32),
        pltpu.VMEM_SHARED((NPAD, EMB), F32),
    ],
)(_edge_body)


_RB = 2048
NB = NPAD // _RB


def _proj_body(x_ref, w_ref, b_ref, h_ref, r_ref):
    h = jnp.dot(x_ref[...], w_ref[...], preferred_element_type=F32) + b_ref[...]
    h_ref[...] = h
    r_ref[...] = jnp.maximum(h, 0.0)


_proj = pl.pallas_call(
    _proj_body,
    grid=(NB,),
    in_specs=[
        pl.BlockSpec((_RB, EMB), lambda i: (i, 0)),
        pl.BlockSpec((EMB, EMB), lambda i: (0, 0)),
        pl.BlockSpec((1, EMB), lambda i: (0, 0)),
    ],
    out_specs=[pl.BlockSpec((_RB, EMB), lambda i: (i, 0))] * 2,
    out_shape=[jax.ShapeDtypeStruct((NPAD, EMB), F32)] * 2,
)


def _gin(hp_ref, a0_ref, a1_ref, epsb_ref, w1_ref, b1_ref, g1_ref, be1_ref,
         w2_ref, b2_ref, g_ref, b_ref):
    y = hp_ref[...] * epsb_ref[...] + (a0_ref[...] + a1_ref[...])
    t = jnp.dot(y, w1_ref[...], preferred_element_type=F32) + b1_ref[...]
    t = jnp.maximum(t * g1_ref[...] + be1_ref[...], 0.0)
    t = jnp.dot(t, w2_ref[...], preferred_element_type=F32) + b2_ref[...]
    return t * g_ref[...] + b_ref[...]


_GIN_SPECS = [
    pl.BlockSpec((_RB, EMB), lambda i: (i, 0)),
    pl.BlockSpec((_RB, EMB), lambda i: (i, 0)),
    pl.BlockSpec((_RB, EMB), lambda i: (i, 0)),
    pl.BlockSpec((1, EMB), lambda i: (0, 0)),
    pl.BlockSpec((EMB, 2 * EMB), lambda i: (0, 0)),
    pl.BlockSpec((1, 2 * EMB), lambda i: (0, 0)),
    pl.BlockSpec((1, 2 * EMB), lambda i: (0, 0)),
    pl.BlockSpec((1, 2 * EMB), lambda i: (0, 0)),
    pl.BlockSpec((2 * EMB, EMB), lambda i: (0, 0)),
    pl.BlockSpec((1, EMB), lambda i: (0, 0)),
    pl.BlockSpec((1, EMB), lambda i: (0, 0)),
    pl.BlockSpec((1, EMB), lambda i: (0, 0)),
]


def _mlp_mid_body(hp_ref, a0_ref, a1_ref, epsb_ref, w1_ref, b1_ref, g1_ref,
                  be1_ref, w2_ref, b2_ref, g_ref, b_ref, batc_ref, vnn_ref,
                  hpn_ref, rn_ref):
    t = jnp.maximum(_gin(hp_ref, a0_ref, a1_ref, epsb_ref, w1_ref, b1_ref,
                         g1_ref, be1_ref, w2_ref, b2_ref, g_ref, b_ref), 0.0)
    oh = (batc_ref[...] == lax.broadcasted_iota(I32, (1, GPAD), 1)).astype(F32)
    hpn = t + jnp.dot(oh, vnn_ref[...], preferred_element_type=F32)
    hpn_ref[...] = hpn
    rn_ref[...] = jnp.maximum(hpn, 0.0)


_mlp_mid = pl.pallas_call(
    _mlp_mid_body,
    grid=(NB,),
    in_specs=_GIN_SPECS + [
        pl.BlockSpec((_RB, 1), lambda i: (i, 0)),
        pl.BlockSpec((GPAD, EMB), lambda i: (0, 0)),
    ],
    out_specs=[pl.BlockSpec((_RB, EMB), lambda i: (i, 0))] * 2,
    out_shape=[jax.ShapeDtypeStruct((NPAD, EMB), F32)] * 2,
)


_DN0 = (((0,), (0,)), ((), ()))


def _vnmlp_body(hp_ref, batc_ref, vn_ref, w1_ref, b1_ref, g1_ref, be1_ref,
                w2_ref, b2_ref, g2_ref, be2_ref, o_ref, acc_ref, cnt_ref):
    i = pl.program_id(0)

    @pl.when(i == 0)
    def _():
        acc_ref[...] = jnp.zeros_like(acc_ref)
        cnt_ref[...] = jnp.zeros_like(cnt_ref)

    oh = (batc_ref[...] == lax.broadcasted_iota(I32, (1, GPAD), 1)).astype(F32)
    acc_ref[...] += lax.dot_general(oh, hp_ref[...], _DN0,
                                    preferred_element_type=F32)
    cnt_ref[...] += lax.dot_general(oh, jnp.ones((_RB, 1), F32), _DN0,
                                    preferred_element_type=F32)

    @pl.when(i == NB - 1)
    def _():
        vn = vn_ref[...]
        vtmp = acc_ref[...] - cnt_ref[...] * vn + vn
        u = jnp.dot(vtmp, w1_ref[...], preferred_element_type=F32) + b1_ref[...]
        u = jnp.maximum(u * g1_ref[...] + be1_ref[...], 0.0)
        u = jnp.dot(u, w2_ref[...], preferred_element_type=F32) + b2_ref[...]
        u = u * g2_ref[...] + be2_ref[...]
        o_ref[...] = jnp.maximum(u, 0.0)


_vnmlp = pl.pallas_call(
    _vnmlp_body,
    grid=(NB,),
    in_specs=[
        pl.BlockSpec((_RB, EMB), lambda i: (i, 0)),
        pl.BlockSpec((_RB, 1), lambda i: (i, 0)),
        pl.BlockSpec((GPAD, EMB), lambda i: (0, 0)),
        pl.BlockSpec((EMB, 2 * EMB), lambda i: (0, 0)),
        pl.BlockSpec((1, 2 * EMB), lambda i: (0, 0)),
        pl.BlockSpec((1, 2 * EMB), lambda i: (0, 0)),
        pl.BlockSpec((1, 2 * EMB), lambda i: (0, 0)),
        pl.BlockSpec((2 * EMB, EMB), lambda i: (0, 0)),
        pl.BlockSpec((1, EMB), lambda i: (0, 0)),
        pl.BlockSpec((1, EMB), lambda i: (0, 0)),
        pl.BlockSpec((1, EMB), lambda i: (0, 0)),
    ],
    out_specs=pl.BlockSpec((GPAD, EMB), lambda i: (0, 0)),
    out_shape=jax.ShapeDtypeStruct((GPAD, EMB), F32),
    scratch_shapes=[
        pltpu.VMEM((GPAD, EMB), F32),
        pltpu.VMEM((GPAD, 1), F32),
    ],
)


def _mlp_head_body(hp_ref, a0_ref, a1_ref, epsb_ref, w1_ref, b1_ref, g1_ref,
                   be1_ref, w2_ref, b2_ref, g_ref, b_ref, batc_ref, lg_ref,
                   lb_ref, wp_ref, bp_ref, out_ref, ge_ref, acc_ref, cnt_ref):
    i = pl.program_id(0)

    @pl.when(i == 0)
    def _():
        acc_ref[...] = jnp.zeros_like(acc_ref)
        cnt_ref[...] = jnp.zeros_like(cnt_ref)

    t = _gin(hp_ref, a0_ref, a1_ref, epsb_ref, w1_ref, b1_ref, g1_ref,
             be1_ref, w2_ref, b2_ref, g_ref, b_ref)
    oh = (batc_ref[...] ==
          lax.broadcasted_iota(I32, (1, NUM_GRAPHS), 1)).astype(F32)
    acc_ref[...] += lax.dot_general(oh, t, _DN0, preferred_element_type=F32)
    cnt_ref[...] += lax.dot_general(oh, jnp.ones((_RB, 1), F32), _DN0,
                                    preferred_element_type=F32)

    @pl.when(i == NB - 1)
    def _():
        ge = acc_ref[...] / jnp.maximum(cnt_ref[...], 1.0)
        mu = jnp.mean(ge, axis=1, keepdims=True)
        var = jnp.mean((ge - mu) ** 2, axis=1, keepdims=True)
        ge = (ge - mu) / jnp.sqrt(var + 1e-5) * lg_ref[...] + lb_ref[...]
        ge_ref[...] = ge
        out_ref[...] = (jnp.dot(ge, wp_ref[...], preferred_element_type=F32)
                        + bp_ref[...])


_mlp_head = pl.pallas_call(
    _mlp_head_body,
    grid=(NB,),
    in_specs=_GIN_SPECS + [
        pl.BlockSpec((_RB, 1), lambda i: (i, 0)),
        pl.BlockSpec((1, EMB), lambda i: (0, 0)),
        pl.BlockSpec((1, EMB), lambda i: (0, 0)),
        pl.BlockSpec((EMB, NUM_CLASS), lambda i: (0, 0)),
        pl.BlockSpec((1, NUM_CLASS), lambda i: (0, 0)),
    ],
    out_specs=[pl.BlockSpec((NUM_GRAPHS, NUM_CLASS), lambda i: (0, 0)),
               pl.BlockSpec((NUM_GRAPHS, EMB), lambda i: (0, 0))],
    out_shape=[jax.ShapeDtypeStruct((NUM_GRAPHS, NUM_CLASS), F32),
               jax.ShapeDtypeStruct((NUM_GRAPHS, EMB), F32)],
    scratch_shapes=[
        pltpu.VMEM((NUM_GRAPHS, EMB), F32),
        pltpu.VMEM((NUM_GRAPHS, 1), F32),
    ],
)


def _row(v):
    return v.reshape(1, -1).astype(F32)


def kernel(x, edge_index, batch, params):
    xp = jnp.pad(x, ((0, NPAD - N_NODES), (0, 0)))
    batp = jnp.pad(batch.astype(I32), (0, NPAD - N_NODES),
                   constant_values=NUM_GRAPHS)
    batc = batp.reshape(NPAD, 1)

    def _slab(a, padval):
        ap = jnp.pad(a.astype(I32), (0, EPAD - N_EDGES),
                     constant_values=padval)
        fa = ap[:EPAD_F].reshape(NS, NCHF, CH)
        sa = jnp.pad(ap[EPAD_F:].reshape(NS, NCHS, CH),
                     ((0, 0), (0, NCHF - NCHS), (0, 0)),
                     constant_values=padval)
        parts = (fa, sa) if FAST_CID == 0 else (sa, fa)
        return jnp.concatenate(parts, axis=0)

    src_slab = _slab(edge_index[0], 0)
    dst_slab = _slab(edge_index[1], JUNK_ROW)

    hp, r = _proj(xp, params['Win'], _row(params['bin']))
    vn = jnp.zeros((GPAD, EMB), F32)

    for l in range(NUM_LAYER):
        p = params['gin%d' % l]
        q = params['bn%d' % l]
        agg = _edge_agg(r, src_slab, dst_slab)
        epsb = (1.0 + p['eps']) * jnp.ones((1, EMB), F32)
        gin_args = (hp, agg[0], agg[1], epsb, p['W1'], _row(p['b1']),
                    _row(p['g1']), _row(p['be1']), p['W2'], _row(p['b2']),
                    _row(q['g']), _row(q['b']))
        if l < NUM_LAYER - 1:
            v = params['vn%d' % l]
            vn = _vnmlp(hp, batc, vn, v['W1'], _row(v['b1']),
                        _row(v['g1']), _row(v['be1']), v['W2'], _row(v['b2']),
                        _row(v['g2']), _row(v['be2']))
            hp, r = _mlp_mid(*gin_args, batc, vn)
        else:
            out, ge = _mlp_head(*gin_args, batc, _row(params['ln']['g']),
                                _row(params['ln']['b']), params['Wp'],
                                _row(params['bp']))
    return out, ge


# speedup vs baseline: 2.1440x; 1.0146x over previous
import functools

import jax
import jax.numpy as jnp
from jax import lax
from jax.experimental import pallas as pl
from jax.experimental.pallas import tpu as pltpu
from jax.experimental.pallas import tpu_sc as plsc

F32 = jnp.float32
I32 = jnp.int32

N_NODES = 10000
N_EDGES = 320000
EMB = 128
NUM_CLASS = 128
NUM_LAYER = 5
NUM_GRAPHS = 512

NC, NS = 2, 16
NW = NC * NS
NPAD = 10240
GPAD = 640
JUNK_ROW = NPAD - 1

CH = 128
FAST_CID = 0
NCHF = 93
NCHS = 64
EPAD_F = NS * NCHF * CH
EPAD_S = NS * NCHS * CH
EPAD = EPAD_F + EPAD_S

NROWS = NPAD // NS

_mesh = plsc.VectorSubcoreMesh(core_axis_name="c", subcore_axis_name="s")


def _edge_body(r_hbm, src_hbm, dst_hbm, z_hbm, out_hbm, src_v, dst_v, buf, agg_sh):
    cid = lax.axis_index("c")
    sid = lax.axis_index("s")
    wid = cid * NS + sid
    nch = jnp.where(cid == FAST_CID, NCHF, NCHS)

    pltpu.sync_copy(src_hbm.at[wid], src_v)
    pltpu.sync_copy(dst_hbm.at[wid], dst_v)

    pltpu.sync_copy(z_hbm, buf)
    def zcopy(i, _):
        pltpu.sync_copy(buf, agg_sh.at[pl.ds(sid * NROWS + i * CH, CH)])
        return 0
    lax.fori_loop(0, NROWS // CH, zcopy, 0)
    plsc.subcore_barrier()

    def chunk(j, _):
        pltpu.sync_copy(r_hbm.at[src_v.at[j]], buf)
        pltpu.sync_copy(buf, agg_sh.at[dst_v.at[j]], add=True)
        return 0
    lax.fori_loop(0, nch, chunk, 0)
    plsc.subcore_barrier()

    pltpu.sync_copy(agg_sh.at[pl.ds(sid * NROWS, NROWS)],
                    out_hbm.at[cid, pl.ds(sid * NROWS, NROWS)])


_edge_agg = functools.partial(
    pl.kernel,
    out_type=jax.ShapeDtypeStruct((NC, NPAD, EMB), F32),
    mesh=_mesh,
    scratch_types=[
        pltpu.VMEM((NCHF, CH), I32),
        pltpu.VMEM((NCHF, CH), I32),
        pltpu.VMEM((CH, EMB), F32),
        pltpu.VMEM_SHARED((NPAD, EMB), F32),
    ],
)(_edge_body)


_RB = 2048
NB = NPAD // _RB


def _proj_body(x_ref, w_ref, b_ref, h_ref, r_ref):
    h = jnp.dot(x_ref[...], w_ref[...], preferred_element_type=F32) + b_ref[...]
    h_ref[...] = h
    r_ref[...] = jnp.maximum(h, 0.0)


_proj = pl.pallas_call(
    _proj_body,
    grid=(NB,),
    in_specs=[
        pl.BlockSpec((_RB, EMB), lambda i: (i, 0)),
        pl.BlockSpec((EMB, EMB), lambda i: (0, 0)),
        pl.BlockSpec((1, EMB), lambda i: (0, 0)),
    ],
    out_specs=[pl.BlockSpec((_RB, EMB), lambda i: (i, 0))] * 2,
    out_shape=[jax.ShapeDtypeStruct((NPAD, EMB), F32)] * 2,
)


def _gin(hp_ref, a0_ref, a1_ref, epsb_ref, w1_ref, b1_ref, g1_ref, be1_ref,
         w2_ref, b2_ref, g_ref, b_ref):
    y = hp_ref[...] * epsb_ref[...] + (a0_ref[...] + a1_ref[...])
    t = jnp.dot(y, w1_ref[...], preferred_element_type=F32) + b1_ref[...]
    t = jnp.maximum(t * g1_ref[...] + be1_ref[...], 0.0)
    t = jnp.dot(t, w2_ref[...], preferred_element_type=F32) + b2_ref[...]
    return t * g_ref[...] + b_ref[...]


_GIN_SPECS = [
    pl.BlockSpec((_RB, EMB), lambda i: (i, 0)),
    pl.BlockSpec((_RB, EMB), lambda i: (i, 0)),
    pl.BlockSpec((_RB, EMB), lambda i: (i, 0)),
    pl.BlockSpec((1, EMB), lambda i: (0, 0)),
    pl.BlockSpec((EMB, 2 * EMB), lambda i: (0, 0)),
    pl.BlockSpec((1, 2 * EMB), lambda i: (0, 0)),
    pl.BlockSpec((1, 2 * EMB), lambda i: (0, 0)),
    pl.BlockSpec((1, 2 * EMB), lambda i: (0, 0)),
    pl.BlockSpec((2 * EMB, EMB), lambda i: (0, 0)),
    pl.BlockSpec((1, EMB), lambda i: (0, 0)),
    pl.BlockSpec((1, EMB), lambda i: (0, 0)),
    pl.BlockSpec((1, EMB), lambda i: (0, 0)),
]


def _mlp_mid_body(hp_ref, a0_ref, a1_ref, epsb_ref, w1_ref, b1_ref, g1_ref,
                  be1_ref, w2_ref, b2_ref, g_ref, b_ref, batc_ref, vnn_ref,
                  hpn_ref, rn_ref):
    t = jnp.maximum(_gin(hp_ref, a0_ref, a1_ref, epsb_ref, w1_ref, b1_ref,
                         g1_ref, be1_ref, w2_ref, b2_ref, g_ref, b_ref), 0.0)
    oh = (batc_ref[...] == lax.broadcasted_iota(I32, (1, GPAD), 1)).astype(F32)
    hpn = t + jnp.dot(oh, vnn_ref[...], preferred_element_type=F32)
    hpn_ref[...] = hpn
    rn_ref[...] = jnp.maximum(hpn, 0.0)


_mlp_mid = pl.pallas_call(
    _mlp_mid_body,
    grid=(NB,),
    in_specs=_GIN_SPECS + [
        pl.BlockSpec((_RB, 1), lambda i: (i, 0)),
        pl.BlockSpec((GPAD, EMB), lambda i: (0, 0)),
    ],
    out_specs=[pl.BlockSpec((_RB, EMB), lambda i: (i, 0))] * 2,
    out_shape=[jax.ShapeDtypeStruct((NPAD, EMB), F32)] * 2,
)


_DN0 = (((0,), (0,)), ((), ()))


def _vnmlp_body(hp_ref, batc_ref, vn_ref, w1_ref, b1_ref, g1_ref, be1_ref,
                w2_ref, b2_ref, g2_ref, be2_ref, o_ref, acc_ref, cnt_ref):
    i = pl.program_id(0)

    @pl.when(i == 0)
    def _():
        acc_ref[...] = jnp.zeros_like(acc_ref)
        cnt_ref[...] = jnp.zeros_like(cnt_ref)

    oh = (batc_ref[...] == lax.broadcasted_iota(I32, (1, GPAD), 1)).astype(F32)
    acc_ref[...] += lax.dot_general(oh, hp_ref[...], _DN0,
                                    preferred_element_type=F32)
    cnt_ref[...] += lax.dot_general(oh, jnp.ones((_RB, 1), F32), _DN0,
                                    preferred_element_type=F32)

    @pl.when(i == NB - 1)
    def _():
        vn = vn_ref[...]
        vtmp = acc_ref[...] - cnt_ref[...] * vn + vn
        u = jnp.dot(vtmp, w1_ref[...], preferred_element_type=F32) + b1_ref[...]
        u = jnp.maximum(u * g1_ref[...] + be1_ref[...], 0.0)
        u = jnp.dot(u, w2_ref[...], preferred_element_type=F32) + b2_ref[...]
        u = u * g2_ref[...] + be2_ref[...]
        o_ref[...] = jnp.maximum(u, 0.0)


_vnmlp = pl.pallas_call(
    _vnmlp_body,
    grid=(NB,),
    in_specs=[
        pl.BlockSpec((_RB, EMB), lambda i: (i, 0)),
        pl.BlockSpec((_RB, 1), lambda i: (i, 0)),
        pl.BlockSpec((GPAD, EMB), lambda i: (0, 0)),
        pl.BlockSpec((EMB, 2 * EMB), lambda i: (0, 0)),
        pl.BlockSpec((1, 2 * EMB), lambda i: (0, 0)),
        pl.BlockSpec((1, 2 * EMB), lambda i: (0, 0)),
        pl.BlockSpec((1, 2 * EMB), lambda i: (0, 0)),
        pl.BlockSpec((2 * EMB, EMB), lambda i: (0, 0)),
        pl.BlockSpec((1, EMB), lambda i: (0, 0)),
        pl.BlockSpec((1, EMB), lambda i: (0, 0)),
        pl.BlockSpec((1, EMB), lambda i: (0, 0)),
    ],
    out_specs=pl.BlockSpec((GPAD, EMB), lambda i: (0, 0)),
    out_shape=jax.ShapeDtypeStruct((GPAD, EMB), F32),
    scratch_shapes=[
        pltpu.VMEM((GPAD, EMB), F32),
        pltpu.VMEM((GPAD, 1), F32),
    ],
)


def _mlp_head_body(hp_ref, a0_ref, a1_ref, epsb_ref, w1_ref, b1_ref, g1_ref,
                   be1_ref, w2_ref, b2_ref, g_ref, b_ref, batc_ref, lg_ref,
                   lb_ref, wp_ref, bp_ref, out_ref, ge_ref, acc_ref, cnt_ref):
    i = pl.program_id(0)

    @pl.when(i == 0)
    def _():
        acc_ref[...] = jnp.zeros_like(acc_ref)
        cnt_ref[...] = jnp.zeros_like(cnt_ref)

    t = _gin(hp_ref, a0_ref, a1_ref, epsb_ref, w1_ref, b1_ref, g1_ref,
             be1_ref, w2_ref, b2_ref, g_ref, b_ref)
    oh = (batc_ref[...] ==
          lax.broadcasted_iota(I32, (1, NUM_GRAPHS), 1)).astype(F32)
    acc_ref[...] += lax.dot_general(oh, t, _DN0, preferred_element_type=F32)
    cnt_ref[...] += lax.dot_general(oh, jnp.ones((_RB, 1), F32), _DN0,
                                    preferred_element_type=F32)

    @pl.when(i == NB - 1)
    def _():
        ge = acc_ref[...] / jnp.maximum(cnt_ref[...], 1.0)
        mu = jnp.mean(ge, axis=1, keepdims=True)
        var = jnp.mean((ge - mu) ** 2, axis=1, keepdims=True)
        ge = (ge - mu) / jnp.sqrt(var + 1e-5) * lg_ref[...] + lb_ref[...]
        ge_ref[...] = ge
        out_ref[...] = (jnp.dot(ge, wp_ref[...], preferred_element_type=F32)
                        + bp_ref[...])


_mlp_head = pl.pallas_call(
    _mlp_head_body,
    grid=(NB,),
    in_specs=_GIN_SPECS + [
        pl.BlockSpec((_RB, 1), lambda i: (i, 0)),
        pl.BlockSpec((1, EMB), lambda i: (0, 0)),
        pl.BlockSpec((1, EMB), lambda i: (0, 0)),
        pl.BlockSpec((EMB, NUM_CLASS), lambda i: (0, 0)),
        pl.BlockSpec((1, NUM_CLASS), lambda i: (0, 0)),
    ],
    out_specs=[pl.BlockSpec((NUM_GRAPHS, NUM_CLASS), lambda i: (0, 0)),
               pl.BlockSpec((NUM_GRAPHS, EMB), lambda i: (0, 0))],
    out_shape=[jax.ShapeDtypeStruct((NUM_GRAPHS, NUM_CLASS), F32),
               jax.ShapeDtypeStruct((NUM_GRAPHS, EMB), F32)],
    scratch_shapes=[
        pltpu.VMEM((NUM_GRAPHS, EMB), F32),
        pltpu.VMEM((NUM_GRAPHS, 1), F32),
    ],
)


def _row(v):
    return v.reshape(1, -1).astype(F32)


def kernel(x, edge_index, batch, params):
    xp = jnp.pad(x, ((0, NPAD - N_NODES), (0, 0)))
    batp = jnp.pad(batch.astype(I32), (0, NPAD - N_NODES),
                   constant_values=NUM_GRAPHS)
    batc = batp.reshape(NPAD, 1)

    def _slab(a, padval):
        ap = jnp.pad(a.astype(I32), (0, EPAD - N_EDGES),
                     constant_values=padval)
        fa = ap[:EPAD_F].reshape(NS, NCHF, CH)
        sa = jnp.pad(ap[EPAD_F:].reshape(NS, NCHS, CH),
                     ((0, 0), (0, NCHF - NCHS), (0, 0)),
                     constant_values=padval)
        parts = (fa, sa) if FAST_CID == 0 else (sa, fa)
        return jnp.concatenate(parts, axis=0)

    src_slab = _slab(edge_index[0], 0)
    zc = jnp.zeros((CH, EMB), F32)
    dst_slab = _slab(edge_index[1], JUNK_ROW)

    hp, r = _proj(xp, params['Win'], _row(params['bin']))
    vn = jnp.zeros((GPAD, EMB), F32)

    for l in range(NUM_LAYER):
        p = params['gin%d' % l]
        q = params['bn%d' % l]
        agg = _edge_agg(r, src_slab, dst_slab, zc)
        epsb = (1.0 + p['eps']) * jnp.ones((1, EMB), F32)
        gin_args = (hp, agg[0], agg[1], epsb, p['W1'], _row(p['b1']),
                    _row(p['g1']), _row(p['be1']), p['W2'], _row(p['b2']),
                    _row(q['g']), _row(q['b']))
        if l < NUM_LAYER - 1:
            v = params['vn%d' % l]
            vn = _vnmlp(hp, batc, vn, v['W1'], _row(v['b1']),
                        _row(v['g1']), _row(v['be1']), v['W2'], _row(v['b2']),
                        _row(v['g2']), _row(v['be2']))
            hp, r = _mlp_mid(*gin_args, batc, vn)
        else:
            out, ge = _mlp_head(*gin_args, batc, _row(params['ln']['g']),
                                _row(params['ln']['b']), params['Wp'],
                                _row(params['bp']))
    return out, ge
